# Initial kernel scaffold; baseline (speedup 1.0000x reference)
import jax
import jax.numpy as jnp
from jax.experimental import pallas as pl


def kernel(x, W1, b1, W2, b2, W3, b3, W4, b4):
    raise NotImplementedError("write your pallas kernel here")


# speedup vs baseline: 42.7058x; 42.7058x over previous
import functools

import jax
import jax.numpy as jnp
from jax import lax
from jax.experimental import pallas as pl
from jax.experimental.pallas import tpu as pltpu
from jax.experimental.pallas import tpu_sc as plsc

N = 8192
H = 128
KNN = 4
ROW_BLK = 64
DEC_BLK = 1024


def _encoder_body(x_ref, w1_ref, b1_ref, w2_ref, b2_ref, h_ref, sq_ref):
    h1 = jnp.maximum(jnp.dot(x_ref[...], w1_ref[...]) + b1_ref[...], 0.0)
    h = jnp.dot(h1, w2_ref[...]) + b2_ref[...]
    h_ref[...] = h
    sq_ref[...] = jnp.sum(h * h, axis=1, keepdims=True)


def _topk_body(hr_ref, hall_ref, sqt_ref, idx_ref):
    p = lax.dot_general(hr_ref[...], hall_ref[...],
                        (((1,), (1,)), ((), ())),
                        preferred_element_type=jnp.float32)
    vals = sqt_ref[...] - 2.0 * p
    r = vals.shape[0]
    iota = lax.broadcasted_iota(jnp.int32, (r, N), 1)
    cols = []
    for k in range(KNN):
        m = jnp.min(vals, axis=1, keepdims=True)
        am = jnp.min(jnp.where(vals == m, iota, jnp.int32(N)),
                     axis=1, keepdims=True)
        cols.append(am)
        if k + 1 < KNN:
            vals = jnp.where(iota == am, jnp.float32(jnp.inf), vals)
    idx_ref[...] = jnp.concatenate(cols, axis=1)


def _decoder_body(z4_ref, w3_ref, b3_ref, w4_ref, b4_ref, zo_ref):
    z0 = z4_ref[:, 0 * H:1 * H]
    z1 = z4_ref[:, 1 * H:2 * H]
    z2 = z4_ref[:, 2 * H:3 * H]
    z3 = z4_ref[:, 3 * H:4 * H]
    mu = (z0 + z1 + z2 + z3) * 0.25
    mx = jnp.maximum(jnp.maximum(z0, z1), jnp.maximum(z2, z3))
    zc = jnp.concatenate([mu, mx], axis=1)
    a1 = jnp.maximum(jnp.dot(zc, w3_ref[...]) + b3_ref[...], 0.0)
    zo_ref[...] = jnp.dot(a1, w4_ref[...]) + b4_ref[...]


def _sc_gather(h, idx_flat):
    info = plsc.get_sparse_core_info()
    nc, ns = info.num_cores, info.num_subcores
    nw = nc * ns
    b = idx_flat.shape[0]
    b_per_w = b // nw
    ch = min(b_per_w, 512)
    nch = b_per_w // ch
    mesh = plsc.VectorSubcoreMesh(core_axis_name="c", subcore_axis_name="s")

    @functools.partial(
        pl.kernel, mesh=mesh,
        out_type=jax.ShapeDtypeStruct((b, H), jnp.float32),
        scratch_types=[
            pltpu.VMEM((ch,), jnp.int32),
            pltpu.VMEM((ch, H), jnp.float32),
            pltpu.SemaphoreType.DMA,
        ],
    )
    def gather_k(h_hbm, idx_hbm, out_hbm, idx_v, rows_v, sem):
        wid = lax.axis_index("s") * nc + lax.axis_index("c")
        for c in range(nch):
            base = wid * b_per_w + c * ch
            pltpu.sync_copy(idx_hbm.at[pl.ds(base, ch)], idx_v)
            pltpu.async_copy(h_hbm.at[idx_v], rows_v, sem).wait()
            pltpu.sync_copy(rows_v, out_hbm.at[pl.ds(base, ch)])

    return gather_k(h, idx_flat)


def kernel(x, W1, b1, W2, b2, W3, b3, W4, b4):
    h, sq = pl.pallas_call(
        _encoder_body,
        out_shape=(
            jax.ShapeDtypeStruct((N, H), jnp.float32),
            jax.ShapeDtypeStruct((N, 1), jnp.float32),
        ),
    )(x, W1, b1.reshape(1, H), W2, b2.reshape(1, H))

    sqt = sq.reshape(1, N)

    idx = pl.pallas_call(
        _topk_body,
        grid=(N // ROW_BLK,),
        in_specs=[
            pl.BlockSpec((ROW_BLK, H), lambda i: (i, 0)),
            pl.BlockSpec((N, H), lambda i: (0, 0)),
            pl.BlockSpec((1, N), lambda i: (0, 0)),
        ],
        out_specs=pl.BlockSpec((ROW_BLK, KNN), lambda i: (i, 0)),
        out_shape=jax.ShapeDtypeStruct((N, KNN), jnp.int32),
    )(h, h, sqt)

    z = _sc_gather(h, idx.reshape(N * KNN))
    z4 = z.reshape(N, KNN * H)

    zo = pl.pallas_call(
        _decoder_body,
        grid=(N // DEC_BLK,),
        in_specs=[
            pl.BlockSpec((DEC_BLK, KNN * H), lambda i: (i, 0)),
            pl.BlockSpec((2 * H, 2 * H), lambda i: (0, 0)),
            pl.BlockSpec((1, 2 * H), lambda i: (0, 0)),
            pl.BlockSpec((2 * H, H), lambda i: (0, 0)),
            pl.BlockSpec((1, H), lambda i: (0, 0)),
        ],
        out_specs=pl.BlockSpec((DEC_BLK, H), lambda i: (i, 0)),
        out_shape=jax.ShapeDtypeStruct((N, H), jnp.float32),
    )(z4, W3, b3.reshape(1, 2 * H), W4, b4.reshape(1, H))

    return (zo[:, :H // 2], zo[:, H // 2:], idx)


# speedup vs baseline: 53.1134x; 1.2437x over previous
import functools

import jax
import jax.numpy as jnp
from jax import lax
from jax.experimental import pallas as pl
from jax.experimental.pallas import tpu as pltpu
from jax.experimental.pallas import tpu_sc as plsc

N = 8192
H = 128
KNN = 4
ROW_BLK = 128
DEC_BLK = 1024


def _encoder_body(x_ref, w1_ref, b1_ref, w2_ref, b2_ref, h_ref, sq_ref):
    h1 = jnp.maximum(jnp.dot(x_ref[...], w1_ref[...]) + b1_ref[...], 0.0)
    h = jnp.dot(h1, w2_ref[...]) + b2_ref[...]
    h_ref[...] = h
    sq_ref[...] = jnp.sum(h * h, axis=1, keepdims=True)


def _topk_body(hr_ref, hall_ref, sqt_ref, idx_ref):
    p = lax.dot_general(hr_ref[...], hall_ref[...],
                        (((1,), (1,)), ((), ())),
                        preferred_element_type=jnp.float32)
    vals = sqt_ref[...] - 2.0 * p
    r = vals.shape[0]
    iota = lax.broadcasted_iota(jnp.int32, (r, N), 1)
    cols = []
    for k in range(KNN):
        m = jnp.min(vals, axis=1, keepdims=True)
        am = jnp.min(jnp.where(vals == m, iota, jnp.int32(N)),
                     axis=1, keepdims=True)
        cols.append(am)
        if k + 1 < KNN:
            vals = jnp.where(iota == am, jnp.float32(jnp.inf), vals)
    idx_ref[...] = jnp.concatenate(cols, axis=1)


def _decoder_body(z4_ref, w3_ref, b3_ref, w4_ref, b4_ref, zo_ref):
    z0 = z4_ref[:, 0 * H:1 * H]
    z1 = z4_ref[:, 1 * H:2 * H]
    z2 = z4_ref[:, 2 * H:3 * H]
    z3 = z4_ref[:, 3 * H:4 * H]
    mu = (z0 + z1 + z2 + z3) * 0.25
    mx = jnp.maximum(jnp.maximum(z0, z1), jnp.maximum(z2, z3))
    zc = jnp.concatenate([mu, mx], axis=1)
    a1 = jnp.maximum(jnp.dot(zc, w3_ref[...]) + b3_ref[...], 0.0)
    zo_ref[...] = jnp.dot(a1, w4_ref[...]) + b4_ref[...]


def _sc_gather(h, idx_flat):
    info = plsc.get_sparse_core_info()
    nc, ns = info.num_cores, info.num_subcores
    nw = nc * ns
    b = idx_flat.shape[0]
    b_per_w = b // nw
    ch = min(b_per_w, 512)
    nch = b_per_w // ch
    mesh = plsc.VectorSubcoreMesh(core_axis_name="c", subcore_axis_name="s")

    @functools.partial(
        pl.kernel, mesh=mesh,
        out_type=jax.ShapeDtypeStruct((b, H), jnp.float32),
        scratch_types=[
            pltpu.VMEM((ch,), jnp.int32),
            pltpu.VMEM((ch, H), jnp.float32),
            pltpu.SemaphoreType.DMA,
        ],
    )
    def gather_k(h_hbm, idx_hbm, out_hbm, idx_v, rows_v, sem):
        wid = lax.axis_index("s") * nc + lax.axis_index("c")
        for c in range(nch):
            base = wid * b_per_w + c * ch
            pltpu.sync_copy(idx_hbm.at[pl.ds(base, ch)], idx_v)
            pltpu.async_copy(h_hbm.at[idx_v], rows_v, sem).wait()
            pltpu.sync_copy(rows_v, out_hbm.at[pl.ds(base, ch)])

    return gather_k(h, idx_flat)


def kernel(x, W1, b1, W2, b2, W3, b3, W4, b4):
    h, sq = pl.pallas_call(
        _encoder_body,
        out_shape=(
            jax.ShapeDtypeStruct((N, H), jnp.float32),
            jax.ShapeDtypeStruct((N, 1), jnp.float32),
        ),
    )(x, W1, b1.reshape(1, H), W2, b2.reshape(1, H))

    sqt = sq.reshape(1, N)

    idx = pl.pallas_call(
        _topk_body,
        grid=(N // ROW_BLK,),
        in_specs=[
            pl.BlockSpec((ROW_BLK, H), lambda i: (i, 0)),
            pl.BlockSpec((N, H), lambda i: (0, 0)),
            pl.BlockSpec((1, N), lambda i: (0, 0)),
        ],
        out_specs=pl.BlockSpec((ROW_BLK, KNN), lambda i: (i, 0)),
        out_shape=jax.ShapeDtypeStruct((N, KNN), jnp.int32),
    )(h, h, sqt)

    z = _sc_gather(h, idx.reshape(N * KNN))
    z4 = z.reshape(N, KNN * H)

    zo = pl.pallas_call(
        _decoder_body,
        grid=(N // DEC_BLK,),
        in_specs=[
            pl.BlockSpec((DEC_BLK, KNN * H), lambda i: (i, 0)),
            pl.BlockSpec((2 * H, 2 * H), lambda i: (0, 0)),
            pl.BlockSpec((1, 2 * H), lambda i: (0, 0)),
            pl.BlockSpec((2 * H, H), lambda i: (0, 0)),
            pl.BlockSpec((1, H), lambda i: (0, 0)),
        ],
        out_specs=pl.BlockSpec((DEC_BLK, H), lambda i: (i, 0)),
        out_shape=jax.ShapeDtypeStruct((N, H), jnp.float32),
    )(z4, W3, b3.reshape(1, 2 * H), W4, b4.reshape(1, H))

    return (zo[:, :H // 2], zo[:, H // 2:], idx)


# speedup vs baseline: 57.0196x; 1.0735x over previous
import functools

import jax
import jax.numpy as jnp
from jax import lax
from jax.experimental import pallas as pl
from jax.experimental.pallas import tpu as pltpu
from jax.experimental.pallas import tpu_sc as plsc

N = 8192
H = 128
KNN = 4
ROW_BLK = 256
DEC_BLK = 1024


def _encoder_body(x_ref, w1_ref, b1_ref, w2_ref, b2_ref, h_ref, sq_ref):
    h1 = jnp.maximum(jnp.dot(x_ref[...], w1_ref[...]) + b1_ref[...], 0.0)
    h = jnp.dot(h1, w2_ref[...]) + b2_ref[...]
    h_ref[...] = h
    sq_ref[...] = jnp.sum(h * h, axis=1, keepdims=True)


def _topk_body(hr_ref, hall_ref, sqt_ref, idx_ref):
    p = lax.dot_general(hr_ref[...], hall_ref[...],
                        (((1,), (1,)), ((), ())),
                        preferred_element_type=jnp.float32)
    vals = sqt_ref[...] - 2.0 * p
    r = vals.shape[0]
    iota = lax.broadcasted_iota(jnp.int32, (r, N), 1)
    cols = []
    for k in range(KNN):
        m = jnp.min(vals, axis=1, keepdims=True)
        am = jnp.min(jnp.where(vals == m, iota, jnp.int32(N)),
                     axis=1, keepdims=True)
        cols.append(am)
        if k + 1 < KNN:
            vals = jnp.where(iota == am, jnp.float32(jnp.inf), vals)
    idx_ref[...] = jnp.concatenate(cols, axis=1)


def _decoder_body(z4_ref, w3_ref, b3_ref, w4_ref, b4_ref, zo_ref):
    z0 = z4_ref[:, 0 * H:1 * H]
    z1 = z4_ref[:, 1 * H:2 * H]
    z2 = z4_ref[:, 2 * H:3 * H]
    z3 = z4_ref[:, 3 * H:4 * H]
    mu = (z0 + z1 + z2 + z3) * 0.25
    mx = jnp.maximum(jnp.maximum(z0, z1), jnp.maximum(z2, z3))
    zc = jnp.concatenate([mu, mx], axis=1)
    a1 = jnp.maximum(jnp.dot(zc, w3_ref[...]) + b3_ref[...], 0.0)
    zo_ref[...] = jnp.dot(a1, w4_ref[...]) + b4_ref[...]


def _sc_gather(h, idx_flat):
    info = plsc.get_sparse_core_info()
    nc, ns = info.num_cores, info.num_subcores
    nw = nc * ns
    b = idx_flat.shape[0]
    b_per_w = b // nw
    ch = min(b_per_w, 512)
    nch = b_per_w // ch
    mesh = plsc.VectorSubcoreMesh(core_axis_name="c", subcore_axis_name="s")

    @functools.partial(
        pl.kernel, mesh=mesh,
        out_type=jax.ShapeDtypeStruct((b, H), jnp.float32),
        scratch_types=[
            pltpu.VMEM((ch,), jnp.int32),
            pltpu.VMEM((ch, H), jnp.float32),
            pltpu.SemaphoreType.DMA,
        ],
    )
    def gather_k(h_hbm, idx_hbm, out_hbm, idx_v, rows_v, sem):
        wid = lax.axis_index("s") * nc + lax.axis_index("c")
        for c in range(nch):
            base = wid * b_per_w + c * ch
            pltpu.sync_copy(idx_hbm.at[pl.ds(base, ch)], idx_v)
            pltpu.async_copy(h_hbm.at[idx_v], rows_v, sem).wait()
            pltpu.sync_copy(rows_v, out_hbm.at[pl.ds(base, ch)])

    return gather_k(h, idx_flat)


def kernel(x, W1, b1, W2, b2, W3, b3, W4, b4):
    h, sq = pl.pallas_call(
        _encoder_body,
        out_shape=(
            jax.ShapeDtypeStruct((N, H), jnp.float32),
            jax.ShapeDtypeStruct((N, 1), jnp.float32),
        ),
    )(x, W1, b1.reshape(1, H), W2, b2.reshape(1, H))

    sqt = sq.reshape(1, N)

    idx = pl.pallas_call(
        _topk_body,
        grid=(N // ROW_BLK,),
        in_specs=[
            pl.BlockSpec((ROW_BLK, H), lambda i: (i, 0)),
            pl.BlockSpec((N, H), lambda i: (0, 0)),
            pl.BlockSpec((1, N), lambda i: (0, 0)),
        ],
        out_specs=pl.BlockSpec((ROW_BLK, KNN), lambda i: (i, 0)),
        out_shape=jax.ShapeDtypeStruct((N, KNN), jnp.int32),
    )(h, h, sqt)

    z = _sc_gather(h, idx.reshape(N * KNN))
    z4 = z.reshape(N, KNN * H)

    zo = pl.pallas_call(
        _decoder_body,
        grid=(N // DEC_BLK,),
        in_specs=[
            pl.BlockSpec((DEC_BLK, KNN * H), lambda i: (i, 0)),
            pl.BlockSpec((2 * H, 2 * H), lambda i: (0, 0)),
            pl.BlockSpec((1, 2 * H), lambda i: (0, 0)),
            pl.BlockSpec((2 * H, H), lambda i: (0, 0)),
            pl.BlockSpec((1, H), lambda i: (0, 0)),
        ],
        out_specs=pl.BlockSpec((DEC_BLK, H), lambda i: (i, 0)),
        out_shape=jax.ShapeDtypeStruct((N, H), jnp.float32),
    )(z4, W3, b3.reshape(1, 2 * H), W4, b4.reshape(1, H))

    return (zo[:, :H // 2], zo[:, H // 2:], idx)


# speedup vs baseline: 59.0134x; 1.0350x over previous
import functools

import jax
import jax.numpy as jnp
from jax import lax
from jax.experimental import pallas as pl
from jax.experimental.pallas import tpu as pltpu
from jax.experimental.pallas import tpu_sc as plsc

N = 8192
H = 128
KNN = 4
ROW_BLK = 512
DEC_BLK = 1024


def _encoder_body(x_ref, w1_ref, b1_ref, w2_ref, b2_ref, h_ref, sq_ref):
    h1 = jnp.maximum(jnp.dot(x_ref[...], w1_ref[...]) + b1_ref[...], 0.0)
    h = jnp.dot(h1, w2_ref[...]) + b2_ref[...]
    h_ref[...] = h
    sq_ref[...] = jnp.sum(h * h, axis=1, keepdims=True)


def _topk_body(hr_ref, hall_ref, sqt_ref, idx_ref):
    p = lax.dot_general(hr_ref[...], hall_ref[...],
                        (((1,), (1,)), ((), ())),
                        preferred_element_type=jnp.float32)
    vals = sqt_ref[...] - 2.0 * p
    r = vals.shape[0]
    iota = lax.broadcasted_iota(jnp.int32, (r, N), 1)
    cols = []
    for k in range(KNN):
        m = jnp.min(vals, axis=1, keepdims=True)
        am = jnp.min(jnp.where(vals == m, iota, jnp.int32(N)),
                     axis=1, keepdims=True)
        cols.append(am)
        if k + 1 < KNN:
            vals = jnp.where(iota == am, jnp.float32(jnp.inf), vals)
    idx_ref[...] = jnp.concatenate(cols, axis=1)


def _decoder_body(z4_ref, w3_ref, b3_ref, w4_ref, b4_ref, zo_ref):
    z0 = z4_ref[:, 0 * H:1 * H]
    z1 = z4_ref[:, 1 * H:2 * H]
    z2 = z4_ref[:, 2 * H:3 * H]
    z3 = z4_ref[:, 3 * H:4 * H]
    mu = (z0 + z1 + z2 + z3) * 0.25
    mx = jnp.maximum(jnp.maximum(z0, z1), jnp.maximum(z2, z3))
    zc = jnp.concatenate([mu, mx], axis=1)
    a1 = jnp.maximum(jnp.dot(zc, w3_ref[...]) + b3_ref[...], 0.0)
    zo_ref[...] = jnp.dot(a1, w4_ref[...]) + b4_ref[...]


def _sc_gather(h, idx_flat):
    info = plsc.get_sparse_core_info()
    nc, ns = info.num_cores, info.num_subcores
    nw = nc * ns
    b = idx_flat.shape[0]
    b_per_w = b // nw
    ch = min(b_per_w, 512)
    nch = b_per_w // ch
    mesh = plsc.VectorSubcoreMesh(core_axis_name="c", subcore_axis_name="s")

    @functools.partial(
        pl.kernel, mesh=mesh,
        out_type=jax.ShapeDtypeStruct((b, H), jnp.float32),
        scratch_types=[
            pltpu.VMEM((ch,), jnp.int32),
            pltpu.VMEM((ch, H), jnp.float32),
            pltpu.SemaphoreType.DMA,
        ],
    )
    def gather_k(h_hbm, idx_hbm, out_hbm, idx_v, rows_v, sem):
        wid = lax.axis_index("s") * nc + lax.axis_index("c")
        for c in range(nch):
            base = wid * b_per_w + c * ch
            pltpu.sync_copy(idx_hbm.at[pl.ds(base, ch)], idx_v)
            pltpu.async_copy(h_hbm.at[idx_v], rows_v, sem).wait()
            pltpu.sync_copy(rows_v, out_hbm.at[pl.ds(base, ch)])

    return gather_k(h, idx_flat)


def kernel(x, W1, b1, W2, b2, W3, b3, W4, b4):
    h, sq = pl.pallas_call(
        _encoder_body,
        out_shape=(
            jax.ShapeDtypeStruct((N, H), jnp.float32),
            jax.ShapeDtypeStruct((N, 1), jnp.float32),
        ),
    )(x, W1, b1.reshape(1, H), W2, b2.reshape(1, H))

    sqt = sq.reshape(1, N)

    idx = pl.pallas_call(
        _topk_body,
        grid=(N // ROW_BLK,),
        in_specs=[
            pl.BlockSpec((ROW_BLK, H), lambda i: (i, 0)),
            pl.BlockSpec((N, H), lambda i: (0, 0)),
            pl.BlockSpec((1, N), lambda i: (0, 0)),
        ],
        out_specs=pl.BlockSpec((ROW_BLK, KNN), lambda i: (i, 0)),
        out_shape=jax.ShapeDtypeStruct((N, KNN), jnp.int32),
    )(h, h, sqt)

    z = _sc_gather(h, idx.reshape(N * KNN))
    z4 = z.reshape(N, KNN * H)

    zo = pl.pallas_call(
        _decoder_body,
        grid=(N // DEC_BLK,),
        in_specs=[
            pl.BlockSpec((DEC_BLK, KNN * H), lambda i: (i, 0)),
            pl.BlockSpec((2 * H, 2 * H), lambda i: (0, 0)),
            pl.BlockSpec((1, 2 * H), lambda i: (0, 0)),
            pl.BlockSpec((2 * H, H), lambda i: (0, 0)),
            pl.BlockSpec((1, H), lambda i: (0, 0)),
        ],
        out_specs=pl.BlockSpec((DEC_BLK, H), lambda i: (i, 0)),
        out_shape=jax.ShapeDtypeStruct((N, H), jnp.float32),
    )(z4, W3, b3.reshape(1, 2 * H), W4, b4.reshape(1, H))

    return (zo[:, :H // 2], zo[:, H // 2:], idx)
